# baseline (device time: 14487 ns/iter reference)
import jax
import jax.numpy as jnp
from jax import lax
from jax.experimental import pallas as pl
from jax.experimental.pallas import tpu as pltpu

N_DEV = 4
N_CHUNK = 4


def kernel(A, B):
    m, _ = A.shape
    _, n = B.shape
    qn = n // N_DEV
    mc = m // N_CHUNK

    def body(a_ref, b_ref, out_ref,
             part_full, rs_buf, ag_src, ag_buf,
             rs_send, rs_recv, ag_send, ag_recv):
        my = lax.axis_index("i")

        barrier_sem = pltpu.get_barrier_semaphore()
        for d in range(1, N_DEV):
            pl.semaphore_signal(
                barrier_sem, inc=1,
                device_id=((my + d) % N_DEV,),
                device_id_type=pl.DeviceIdType.MESH,
            )

        def start_rs(c):
            sends = []
            for d in (2, 1, 3):
                q = (my + d) % N_DEV
                r = pltpu.make_async_remote_copy(
                    src_ref=part_full.at[pl.ds(c * mc, mc), pl.ds(q * qn, qn)],
                    dst_ref=rs_buf.at[d - 1, c],
                    send_sem=rs_send.at[d - 1, c],
                    recv_sem=rs_recv.at[d - 1, c],
                    device_id=((my + d) % N_DEV,),
                    device_id_type=pl.DeviceIdType.MESH,
                )
                r.start()
                sends.append(r)
            return sends

        def start_ag(c):
            sends = []
            for d in (2, 1, 3):
                r = pltpu.make_async_remote_copy(
                    src_ref=ag_src.at[c],
                    dst_ref=ag_buf.at[d - 1, c],
                    send_sem=ag_send.at[d - 1, c],
                    recv_sem=ag_recv.at[d - 1, c],
                    device_id=((my + d) % N_DEV,),
                    device_id_type=pl.DeviceIdType.MESH,
                )
                r.start()
                sends.append(r)
            return sends

        def reduce_and_ag(c, rs_sends):
            for r in rs_sends:
                r.wait_recv()
            red = part_full[pl.ds(c * mc, mc), pl.ds(my * qn, qn)].astype(
                jnp.float32
            )
            red = red + (
                rs_buf[0, c].astype(jnp.float32)
                + rs_buf[1, c].astype(jnp.float32)
                + rs_buf[2, c].astype(jnp.float32)
            )
            out_ref[pl.ds(c * mc, mc), pl.ds(my * qn, qn)] = red
            ag_src[c] = red.astype(jnp.bfloat16)
            return start_ag(c)

        ab = a_ref[:, :].astype(jnp.bfloat16)
        bb = b_ref[:, :].astype(jnp.bfloat16)

        rs = []
        for c in range(N_CHUNK):
            part_full[c * mc:(c + 1) * mc, :] = jnp.dot(
                ab[c * mc:(c + 1) * mc, :], bb,
                preferred_element_type=jnp.float32,
            ).astype(jnp.bfloat16)
            if c == 0:
                pl.semaphore_wait(barrier_sem, N_DEV - 1)
            rs.append(start_rs(c))

        ag = [reduce_and_ag(c, rs[c]) for c in range(N_CHUNK)]

        for c in range(N_CHUNK):
            for d, r in zip((2, 1, 3), ag[c]):
                r.wait_recv()
                src_dev = (my - d) % N_DEV
                out_ref[pl.ds(c * mc, mc), pl.ds(src_dev * qn, qn)] = (
                    ag_buf[d - 1, c].astype(jnp.float32)
                )

        for sends in rs + ag:
            for r in sends:
                r.wait_send()

    return pl.pallas_call(
        body,
        out_shape=jax.ShapeDtypeStruct((m, n), jnp.float32),
        in_specs=[
            pl.BlockSpec(memory_space=pltpu.VMEM),
            pl.BlockSpec(memory_space=pltpu.VMEM),
        ],
        out_specs=pl.BlockSpec(memory_space=pltpu.VMEM),
        scratch_shapes=[
            pltpu.VMEM((m, n), jnp.bfloat16),
            pltpu.VMEM((N_DEV - 1, N_CHUNK, mc, qn), jnp.bfloat16),
            pltpu.VMEM((N_CHUNK, mc, qn), jnp.bfloat16),
            pltpu.VMEM((N_DEV - 1, N_CHUNK, mc, qn), jnp.bfloat16),
            pltpu.SemaphoreType.DMA((N_DEV - 1, N_CHUNK)),
            pltpu.SemaphoreType.DMA((N_DEV - 1, N_CHUNK)),
            pltpu.SemaphoreType.DMA((N_DEV - 1, N_CHUNK)),
            pltpu.SemaphoreType.DMA((N_DEV - 1, N_CHUNK)),
        ],
        compiler_params=pltpu.CompilerParams(collective_id=0),
    )(A, B)
